# baseline (speedup 1.0000x reference)
import functools

import jax
import jax.numpy as jnp
from jax.experimental import pallas as pl
from jax.experimental.pallas import tpu as pltpu


def _make_body(E, NT, NF, TM, FK, S):
    NTF = NT * NF

    def body(x_ref, wg_ref, bg_ref, w1_ref, b1_ref, w2_ref, b2_ref,
             out_ref, w_scr, h_scr):
        s = pl.program_id(0)

        sp = jnp.maximum(s, 1) - 1
        e_p = sp // NTF
        t_p = (sp // NF) % NT
        f_p = sp % NF
        q = (s + 1) % 2

        y = jnp.dot(h_scr[q], w2_ref[0].astype(jnp.bfloat16),
                    preferred_element_type=jnp.float32)
        tsp = pl.ds(t_p * TM, TM)
        w_tile = w_scr[tsp, :]
        lane = jax.lax.broadcasted_iota(jnp.int32, w_tile.shape, 1)
        col = jnp.sum(jnp.where(lane == e_p, w_tile, 0.0),
                      axis=1, keepdims=True)
        contrib = col * y
        first = (e_p == 0) & (f_p == 0)

        @pl.when(first)
        def _():
            out_ref[tsp, :] = jnp.dot(
                w_tile, b2_ref[...],
                preferred_element_type=jnp.float32) + contrib

        @pl.when(jnp.logical_not(first))
        def _():
            out_ref[tsp, :] += contrib

        e = s // NTF
        t = (s // NF) % NT
        f = s % NF
        x_t = x_ref[...]

        @pl.when((e == 0) & (f == 0))
        def _gate():
            scores = jnp.dot(x_t, wg_ref[...],
                             preferred_element_type=jnp.float32)
            scores = scores + bg_ref[...]
            m = jnp.max(scores, axis=-1, keepdims=True)
            ex = jnp.exp(scores - m)
            w_scr[pl.ds(t * TM, TM), :] = ex / jnp.sum(
                ex, axis=-1, keepdims=True)

        h = jnp.dot(x_t.astype(jnp.bfloat16),
                    w1_ref[0].astype(jnp.bfloat16),
                    preferred_element_type=jnp.float32)
        h = h + b1_ref[0]
        h = 0.5 * h * (1.0 + jax.lax.erf(h * 0.7071067811865476))
        h_scr[s % 2] = h.astype(jnp.bfloat16)

    return body


@functools.partial(jax.jit, static_argnames=())
def kernel(x, Wg, bg, W1, b1, W2, b2):
    T, DIM = x.shape
    E = Wg.shape[1]
    FF = W1.shape[2]
    TM = 1024
    FK = 1024
    NT = T // TM
    NF = FF // FK
    NTF = NT * NF
    S = E * NTF

    bg2 = bg.reshape(1, E)
    b1_3d = b1.reshape(E, 1, FF)

    def prod_ids(s):
        sc = jnp.minimum(s, S - 1)
        return sc // NTF, (sc // NF) % NT, sc % NF

    def cons_ids(s):
        sp = jnp.maximum(s, 1) - 1
        return sp // NTF, (sp // NF) % NT, sp % NF

    body = _make_body(E, NT, NF, TM, FK, S)

    return pl.pallas_call(
        body,
        grid=(S + 1,),
        in_specs=[
            pl.BlockSpec((TM, DIM), lambda s: (prod_ids(s)[1], 0)),
            pl.BlockSpec((DIM, E), lambda s: (0, 0)),
            pl.BlockSpec((1, E), lambda s: (0, 0)),
            pl.BlockSpec((1, DIM, FK),
                         lambda s: (prod_ids(s)[0], 0, prod_ids(s)[2])),
            pl.BlockSpec((1, 1, FK),
                         lambda s: (prod_ids(s)[0], 0, prod_ids(s)[2])),
            pl.BlockSpec((1, FK, DIM),
                         lambda s: (cons_ids(s)[0], cons_ids(s)[2], 0)),
            pl.BlockSpec((E, DIM), lambda s: (0, 0)),
        ],
        out_specs=pl.BlockSpec((T, DIM), lambda s: (0, 0)),
        out_shape=jax.ShapeDtypeStruct((T, DIM), jnp.float32),
        scratch_shapes=[
            pltpu.VMEM((T, E), jnp.float32),
            pltpu.VMEM((2, TM, FK), jnp.bfloat16),
        ],
        compiler_params=pltpu.CompilerParams(
            dimension_semantics=("arbitrary",),
            vmem_limit_bytes=64 * 1024 * 1024,
        ),
    )(x, Wg, bg2, W1, b1_3d, W2, b2)


# speedup vs baseline: 1.1700x; 1.1700x over previous
import functools

import jax
import jax.numpy as jnp
from jax.experimental import pallas as pl
from jax.experimental.pallas import tpu as pltpu


def _gate_body(x_ref, wg_ref, bg_ref, w_ref):
    scores = jnp.dot(x_ref[...], wg_ref[...],
                     preferred_element_type=jnp.float32) + bg_ref[...]
    m = jnp.max(scores, axis=-1, keepdims=True)
    ex = jnp.exp(scores - m)
    w_ref[...] = ex / jnp.sum(ex, axis=-1, keepdims=True)


def _make_main_body(NT, NF, TM, FK):
    NTF = NT * NF

    def body(x_ref, w1_ref, b1_ref, w2_ref, w_ref, b2_ref,
             out_ref, h_scr):
        s = pl.program_id(0)

        @pl.when(s == 0)
        def _():
            h_scr[1] = jnp.zeros_like(h_scr[1])
            out_ref[...] = jnp.dot(w_ref[...], b2_ref[...],
                                   preferred_element_type=jnp.float32)

        sp = s - 1
        e_p = sp // NTF
        t_p = (sp // NF) % NT
        q = (s + 1) % 2

        y = jnp.dot(h_scr[q], w2_ref[0].astype(jnp.bfloat16),
                    preferred_element_type=jnp.float32)
        tsp = pl.ds(t_p * TM, TM)
        w_tile = w_ref[tsp, :]
        lane = jax.lax.broadcasted_iota(jnp.int32, w_tile.shape, 1)
        col = jnp.sum(jnp.where(lane == e_p, w_tile, 0.0),
                      axis=1, keepdims=True)
        out_ref[tsp, :] += col * y

        h = jnp.dot(x_ref[...].astype(jnp.bfloat16),
                    w1_ref[0].astype(jnp.bfloat16),
                    preferred_element_type=jnp.float32)
        h = h + b1_ref[0]
        h = 0.5 * h * (1.0 + jax.lax.erf(h * 0.7071067811865476))
        h_scr[s % 2] = h.astype(jnp.bfloat16)

    return body


@functools.partial(jax.jit, static_argnames=())
def kernel(x, Wg, bg, W1, b1, W2, b2):
    T, DIM = x.shape
    E = Wg.shape[1]
    FF = W1.shape[2]
    TM = 1024
    FK = 1024
    NT = T // TM
    NF = FF // FK
    NTF = NT * NF
    S = E * NTF

    bg2 = bg.reshape(1, E)
    b1_3d = b1.reshape(E, 1, FF)

    w = pl.pallas_call(
        _gate_body,
        grid=(NT,),
        in_specs=[
            pl.BlockSpec((TM, DIM), lambda t: (t, 0)),
            pl.BlockSpec((DIM, E), lambda t: (0, 0)),
            pl.BlockSpec((1, E), lambda t: (0, 0)),
        ],
        out_specs=pl.BlockSpec((TM, E), lambda t: (t, 0)),
        out_shape=jax.ShapeDtypeStruct((T, E), jnp.float32),
    )(x, Wg, bg2)

    def prod_ids(s):
        sc = jnp.minimum(s, S - 1)
        return sc // NTF, (sc // NF) % NT, sc % NF

    def cons_ids(s):
        sp = jnp.maximum(s, 1) - 1
        return sp // NTF, (sp // NF) % NT, sp % NF

    body = _make_main_body(NT, NF, TM, FK)

    return pl.pallas_call(
        body,
        grid=(S + 1,),
        in_specs=[
            pl.BlockSpec((TM, DIM), lambda s: (prod_ids(s)[1], 0)),
            pl.BlockSpec((1, DIM, FK),
                         lambda s: (prod_ids(s)[0], 0, prod_ids(s)[2])),
            pl.BlockSpec((1, 1, FK),
                         lambda s: (prod_ids(s)[0], 0, prod_ids(s)[2])),
            pl.BlockSpec((1, FK, DIM),
                         lambda s: (cons_ids(s)[0], cons_ids(s)[2], 0)),
            pl.BlockSpec((T, E), lambda s: (0, 0)),
            pl.BlockSpec((E, DIM), lambda s: (0, 0)),
        ],
        out_specs=pl.BlockSpec((T, DIM), lambda s: (0, 0)),
        out_shape=jax.ShapeDtypeStruct((T, DIM), jnp.float32),
        scratch_shapes=[
            pltpu.VMEM((2, TM, FK), jnp.bfloat16),
        ],
        compiler_params=pltpu.CompilerParams(
            dimension_semantics=("arbitrary",),
            vmem_limit_bytes=64 * 1024 * 1024,
        ),
    )(x, W1, b1_3d, W2, w, b2)
